# Initial kernel scaffold; baseline (speedup 1.0000x reference)
import jax
import jax.numpy as jnp
from jax.experimental import pallas as pl


def kernel(x, edge_index, edge_attr, W1, b1, K1, Kb1, K2, Kb2, K3, Kb3, root, bias_c, W2, b2):
    raise NotImplementedError("write your pallas kernel here")


# speedup vs baseline: 2.6029x; 2.6029x over previous
import functools

import jax
import jax.numpy as jnp
import numpy as np
from jax import lax
from jax.experimental import pallas as pl
from jax.experimental.pallas import tpu as pltpu
from jax.experimental.pallas import tpu_sc as plsc

N = 10000
E = 160000
W = 32
DEPTH = 6

NC, NS = 2, 16
NW = NC * NS
EP = 163840
EW = EP // NW
CH = 1024
NCHUNK = EW // CH
NP = N + 16
TE = 2048
TN = 2000

_sc_mesh = plsc.VectorSubcoreMesh(core_axis_name="c", subcore_axis_name="s")


@functools.partial(
    pl.kernel,
    out_type=jax.ShapeDtypeStruct((EP, W), jnp.float32),
    mesh=_sc_mesh,
    scratch_types=[
        pltpu.VMEM((8, 128), jnp.int32),
        pltpu.VMEM((CH, W), jnp.float32),
        pltpu.SemaphoreType.DMA,
    ],
    compiler_params=pltpu.CompilerParams(use_tc_tiling_on_sc=False),
)
def _gather(h_hbm, src_hbm, out_hbm, idx_v, rows_v, sem):
    c = lax.axis_index("c")
    s = lax.axis_index("s")
    wid = c * NS + s
    for j in range(NCHUNK):
        r0 = wid * (EW // 128) + j * 8
        pltpu.sync_copy(src_hbm.at[pl.ds(r0, 8)], idx_v)
        handles = []
        for j2 in range(8):
            handles.append(
                pltpu.async_copy(
                    h_hbm.at[idx_v.at[j2]],
                    rows_v.at[pl.ds(j2 * 128, 128)],
                    sem,
                )
            )
        for hd in handles:
            hd.wait()
        pltpu.sync_copy(rows_v, out_hbm.at[pl.ds(wid * EW + j * CH, CH)])


@functools.partial(
    pl.kernel,
    out_type=jax.ShapeDtypeStruct((NC, N, W), jnp.float32),
    mesh=_sc_mesh,
    scratch_types=[
        pltpu.VMEM((8, 128), jnp.int32),
        pltpu.VMEM((CH, W), jnp.float32),
        pltpu.VMEM_SHARED((NP, W), jnp.float32),
        pltpu.SemaphoreType.DMA,
    ],
    compiler_params=pltpu.CompilerParams(use_tc_tiling_on_sc=False),
)
def _scatter(rows_hbm, dst_hbm, zeros_hbm, out_hbm, idx_v, rows_v, aggr_sh, sem):
    c = lax.axis_index("c")
    s = lax.axis_index("s")
    wid = c * NS + s
    pltpu.sync_copy(zeros_hbm.at[pl.ds(s * (NP // NS), NP // NS)],
                    aggr_sh.at[pl.ds(s * (NP // NS), NP // NS)])
    plsc.subcore_barrier()
    for j in range(NCHUNK):
        r0 = wid * (EW // 128) + j * 8
        pltpu.sync_copy(dst_hbm.at[pl.ds(r0, 8)], idx_v)
        pltpu.sync_copy(rows_hbm.at[pl.ds(wid * EW + j * CH, CH)], rows_v)
        for j2 in range(8):
            pltpu.sync_copy(rows_v.at[pl.ds(j2 * 128, 128)],
                            aggr_sh.at[idx_v.at[j2]], add=True)
    plsc.subcore_barrier()
    pltpu.sync_copy(aggr_sh.at[pl.ds(s * (N // NS), N // NS)],
                    out_hbm.at[c, pl.ds(s * (N // NS), N // NS)])


def _bf(v):
    return v.astype(jnp.bfloat16)


def _mlp_body(ea, K1, Kb1, K2, Kb2, o):
    a = jnp.maximum(jnp.dot(_bf(ea[...]), _bf(K1[...]), preferred_element_type=jnp.float32) + Kb1[...], 0.0)
    o[...] = jnp.maximum(jnp.dot(_bf(a), _bf(K2[...]), preferred_element_type=jnp.float32) + Kb2[...], 0.0)


def _edge_mlp(ea, K1, Kb1, K2, Kb2):
    return pl.pallas_call(
        _mlp_body,
        grid=(EP // TE,),
        in_specs=[
            pl.BlockSpec((TE, 4), lambda i: (i, 0)),
            pl.BlockSpec((4, 64), lambda i: (0, 0)),
            pl.BlockSpec((1, 64), lambda i: (0, 0)),
            pl.BlockSpec((64, 128), lambda i: (0, 0)),
            pl.BlockSpec((1, 128), lambda i: (0, 0)),
        ],
        out_specs=pl.BlockSpec((TE, 128), lambda i: (i, 0)),
        out_shape=jax.ShapeDtypeStruct((EP, 128), jnp.float32),
    )(ea, K1, Kb1, K2, Kb2)


def _msg_body(k2, xj, K3, Kb3, R, S, o):
    w = jnp.dot(_bf(k2[...]), _bf(K3[...]), preferred_element_type=jnp.float32) + Kb3[...]
    xe = jnp.dot(_bf(xj[...]), _bf(R[...]), preferred_element_type=jnp.float32)
    prod = xe * _bf(w).astype(jnp.float32)
    y = prod[:, 0:128]
    for g in range(1, 8):
        y = y + prod[:, g * 128:(g + 1) * 128]
    o[...] = (y[:, 0:W] + y[:, W:2 * W]) + (y[:, 2 * W:3 * W] + y[:, 3 * W:4 * W])


def _msg(k2, xj, K3, Kb3, R, S):
    return pl.pallas_call(
        _msg_body,
        grid=(EP // TE,),
        in_specs=[
            pl.BlockSpec((TE, 128), lambda i: (i, 0)),
            pl.BlockSpec((TE, W), lambda i: (i, 0)),
            pl.BlockSpec((128, W * W), lambda i: (0, 0)),
            pl.BlockSpec((1, W * W), lambda i: (0, 0)),
            pl.BlockSpec((W, W * W), lambda i: (0, 0)),
            pl.BlockSpec((128, W), lambda i: (0, 0)),
        ],
        out_specs=pl.BlockSpec((TE, W), lambda i: (i, 0)),
        out_shape=jax.ShapeDtypeStruct((EP, W), jnp.float32),
    )(k2, xj, K3, Kb3, R, S)


def _update_body(a0, a1, invd, h, root, bias, o):
    aggr = (a0[...] + a1[...]) * invd[...]
    o[...] = jnp.maximum(
        aggr + jnp.dot(_bf(h[...]), _bf(root[...]), preferred_element_type=jnp.float32) + bias[...], 0.0)


def _update(a0, a1, invd, h, root, bias):
    return pl.pallas_call(
        _update_body,
        grid=(N // TN,),
        in_specs=[
            pl.BlockSpec((TN, W), lambda i: (i, 0)),
            pl.BlockSpec((TN, W), lambda i: (i, 0)),
            pl.BlockSpec((TN, W), lambda i: (i, 0)),
            pl.BlockSpec((TN, W), lambda i: (i, 0)),
            pl.BlockSpec((W, W), lambda i: (0, 0)),
            pl.BlockSpec((1, W), lambda i: (0, 0)),
        ],
        out_specs=pl.BlockSpec((TN, W), lambda i: (i, 0)),
        out_shape=jax.ShapeDtypeStruct((N, W), jnp.float32),
    )(a0, a1, invd, h, root, bias)


def _invdeg_body(d0, d1, o):
    deg = jnp.maximum(d0[:, 0:1] + d1[:, 0:1], 1.0)
    o[...] = jnp.broadcast_to(1.0 / deg, (TN, W))


def _invdeg(d0, d1):
    return pl.pallas_call(
        _invdeg_body,
        grid=(N // TN,),
        in_specs=[
            pl.BlockSpec((TN, W), lambda i: (i, 0)),
            pl.BlockSpec((TN, W), lambda i: (i, 0)),
        ],
        out_specs=pl.BlockSpec((TN, W), lambda i: (i, 0)),
        out_shape=jax.ShapeDtypeStruct((N, W), jnp.float32),
    )(d0, d1)


def _h0_body(x, W1, b1, o):
    o[...] = x[...] * W1[...] + b1[...]


def _h0(x, W1, b1):
    return pl.pallas_call(
        _h0_body,
        grid=(N // TN,),
        in_specs=[
            pl.BlockSpec((TN, 1), lambda i: (i, 0)),
            pl.BlockSpec((1, W), lambda i: (0, 0)),
            pl.BlockSpec((1, W), lambda i: (0, 0)),
        ],
        out_specs=pl.BlockSpec((TN, W), lambda i: (i, 0)),
        out_shape=jax.ShapeDtypeStruct((N, W), jnp.float32),
    )(x, W1, b1)


def _out_body(h, W2t, b2, o):
    hb = _bf(h[...]).astype(jnp.float32)
    wb = _bf(W2t[...]).astype(jnp.float32)
    o[...] = jnp.sum(hb * wb, axis=1, keepdims=True) + b2[...]


def _final(h, W2t, b2):
    return pl.pallas_call(
        _out_body,
        grid=(N // TN,),
        in_specs=[
            pl.BlockSpec((TN, W), lambda i: (i, 0)),
            pl.BlockSpec((1, W), lambda i: (0, 0)),
            pl.BlockSpec((1, 1), lambda i: (0, 0)),
        ],
        out_specs=pl.BlockSpec((TN, 1), lambda i: (i, 0)),
        out_shape=jax.ShapeDtypeStruct((N, 1), jnp.float32),
    )(h, W2t, b2)


def kernel(x, edge_index, edge_attr, W1, b1, K1, Kb1, K2, Kb2, K3, Kb3, root, bias_c, W2, b2):
    f32 = jnp.float32
    src = edge_index[0]
    dst = edge_index[1]
    src_p = jnp.concatenate([src, jnp.zeros((EP - E,), jnp.int32)]).reshape(EP // 128, 128)
    dst_p = jnp.concatenate([dst, jnp.full((EP - E,), N, jnp.int32)]).reshape(EP // 128, 128)
    ea_p = jnp.pad(edge_attr, ((0, EP - E), (0, 0)))

    Rm = jnp.asarray(np.repeat(np.eye(W, dtype=np.float32), W, axis=1))
    Sm = jnp.asarray(np.tile(np.eye(W, dtype=np.float32), (4, 1)))
    zeros_pad = jnp.zeros((NP, W), f32)
    deg_rows = jnp.broadcast_to(
        jnp.zeros((W,), f32).at[0].set(1.0), (EP, W))

    h = _h0(x, W1.reshape(1, W), b1.reshape(1, W))
    k2 = _edge_mlp(ea_p, K1, Kb1.reshape(1, 64), K2, Kb2.reshape(1, 128))
    degp = _scatter(deg_rows, dst_p, zeros_pad)
    invd = _invdeg(degp[0], degp[1])

    Kb3r = Kb3.reshape(1, W * W)
    biasr = bias_c.reshape(1, W)
    for _ in range(DEPTH):
        xj = _gather(h, src_p)
        msg = _msg(k2, xj, K3, Kb3r, Rm, Sm)
        aggp = _scatter(msg, dst_p, zeros_pad)
        h = _update(aggp[0], aggp[1], invd, h, root, biasr)

    return _final(h, W2.reshape(1, W), b2.reshape(1, 1))
